# Initial kernel scaffold; baseline (speedup 1.0000x reference)
import functools

import jax
import jax.numpy as jnp
from jax import lax
from jax.experimental import pallas as pl
from jax.experimental.pallas import tpu as pltpu
from jax.experimental.pallas import tpu_sc as plsc

POOL_T = 16
N_SEEDS = 4096
HOP1 = N_SEEDS * POOL_T
TOTAL = HOP1 + N_SEEDS

NUM_CORES = 2
NUM_SUBCORES = 16
NUM_WORKERS = NUM_CORES * NUM_SUBCORES
LANES = 16
SEEDS_PER_W = N_SEEDS // NUM_WORKERS


@functools.partial(
    pl.kernel,
    out_type=(
        jax.ShapeDtypeStruct((POOL_T + 1, N_SEEDS, POOL_T), jnp.int32),
        jax.ShapeDtypeStruct((POOL_T + 1, N_SEEDS, POOL_T), jnp.float32),
        jax.ShapeDtypeStruct((N_SEEDS, POOL_T), jnp.float32),
    ),
    mesh=plsc.VectorSubcoreMesh(core_axis_name="c", subcore_axis_name="s"),
    scratch_types=[
        pltpu.VMEM((SEEDS_PER_W,), jnp.int32),
        pltpu.VMEM((SEEDS_PER_W, POOL_T), jnp.int32),
        pltpu.VMEM((SEEDS_PER_W, POOL_T), jnp.float32),
        pltpu.VMEM((POOL_T, SEEDS_PER_W), jnp.int32),
        pltpu.VMEM((POOL_T, SEEDS_PER_W, POOL_T), jnp.int32),
        pltpu.VMEM((POOL_T, SEEDS_PER_W, POOL_T), jnp.float32),
        pltpu.SemaphoreType.DMA,
    ],
)
def _sampler_sc(node_ids_hbm, neigh_hbm, alpha_hbm,
                neigh_out, alpha_out, alpha0_out,
                idx0_v, neigh0_v, alpha0_v, idx1_v, neigh1_v, alpha1_v, sem):
    wid = lax.axis_index("s") * NUM_CORES + lax.axis_index("c")
    base = wid * SEEDS_PER_W

    pltpu.sync_copy(node_ids_hbm.at[pl.ds(base, SEEDS_PER_W)], idx0_v)
    cp_n0 = pltpu.async_copy(neigh_hbm.at[idx0_v], neigh0_v, sem)
    cp_a0 = pltpu.async_copy(alpha_hbm.at[idx0_v], alpha0_v, sem)
    cp_n0.wait()
    cp_a0.wait()

    pltpu.sync_copy(neigh0_v, neigh_out.at[POOL_T, pl.ds(base, SEEDS_PER_W), :])
    pltpu.sync_copy(alpha0_v, alpha_out.at[POOL_T, pl.ds(base, SEEDS_PER_W), :])
    pltpu.sync_copy(alpha0_v, alpha0_out.at[pl.ds(base, SEEDS_PER_W), :])

    for t in range(POOL_T):
        col = jnp.full((LANES,), t, jnp.int32)
        for g in range(SEEDS_PER_W // LANES):
            rows = lax.iota(jnp.int32, LANES) + g * LANES
            idx1_v[t, pl.ds(g * LANES, LANES)] = plsc.load_gather(
                neigh0_v, [rows, col])

    cp_n1 = pltpu.async_copy(neigh_hbm.at[idx1_v], neigh1_v, sem)
    cp_a1 = pltpu.async_copy(alpha_hbm.at[idx1_v], alpha1_v, sem)
    cp_n1.wait()
    cp_a1.wait()

    pltpu.sync_copy(neigh1_v, neigh_out.at[pl.ds(0, POOL_T), pl.ds(base, SEEDS_PER_W), :])
    pltpu.sync_copy(alpha1_v, alpha_out.at[pl.ds(0, POOL_T), pl.ds(base, SEEDS_PER_W), :])


def kernel(node_ids, neigh_table, alpha_table):
    neigh3, alpha3, alpha0 = _sampler_sc(node_ids, neigh_table, alpha_table)
    neigh_all = neigh3.reshape(TOTAL, POOL_T)
    alpha_all = alpha3.reshape(TOTAL, POOL_T)
    ids = jnp.concatenate(
        [jnp.transpose(neigh3[POOL_T]).reshape(-1), node_ids.astype(jnp.int32)])
    stacks = jnp.concatenate([ids[:, None], neigh_all], axis=1)
    return stacks, alpha0, alpha_all


# speedup vs baseline: 1.0763x; 1.0763x over previous
import functools

import jax
import jax.numpy as jnp
from jax import lax
from jax.experimental import pallas as pl
from jax.experimental.pallas import tpu as pltpu
from jax.experimental.pallas import tpu_sc as plsc

POOL_T = 16
N_SEEDS = 4096
HOP1 = N_SEEDS * POOL_T
TOTAL = HOP1 + N_SEEDS

NUM_CORES = 2
NUM_SUBCORES = 16
NUM_WORKERS = NUM_CORES * NUM_SUBCORES
IDX_W = 128

SEEDS_PER_W = N_SEEDS // NUM_WORKERS
H1_PER_W = HOP1 // NUM_WORKERS
H1_CHUNKS = H1_PER_W // IDX_W

_MESH = plsc.VectorSubcoreMesh(core_axis_name="c", subcore_axis_name="s")


@functools.partial(
    pl.kernel,
    out_type=(
        jax.ShapeDtypeStruct((N_SEEDS, POOL_T), jnp.int32),
        jax.ShapeDtypeStruct((N_SEEDS, POOL_T), jnp.float32),
    ),
    mesh=_MESH,
    compiler_params=pltpu.CompilerParams(use_tc_tiling_on_sc=False),
    scratch_types=[
        pltpu.VMEM((SEEDS_PER_W,), jnp.int32),
        pltpu.VMEM((SEEDS_PER_W, POOL_T), jnp.int32),
        pltpu.VMEM((SEEDS_PER_W, POOL_T), jnp.float32),
        pltpu.SemaphoreType.DMA,
    ],
)
def _hop0_sc(node_ids_hbm, neigh_hbm, alpha_hbm, neigh_out, alpha_out,
             idx_v, neigh_v, alpha_v, sem):
    wid = lax.axis_index("s") * NUM_CORES + lax.axis_index("c")
    base = wid * SEEDS_PER_W
    pltpu.sync_copy(node_ids_hbm.at[pl.ds(base, SEEDS_PER_W)], idx_v)
    cp_n = pltpu.async_copy(neigh_hbm.at[idx_v], neigh_v, sem)
    cp_a = pltpu.async_copy(alpha_hbm.at[idx_v], alpha_v, sem)
    cp_n.wait()
    cp_a.wait()
    pltpu.sync_copy(neigh_v, neigh_out.at[pl.ds(base, SEEDS_PER_W), :])
    pltpu.sync_copy(alpha_v, alpha_out.at[pl.ds(base, SEEDS_PER_W), :])


@functools.partial(
    pl.kernel,
    out_type=(
        jax.ShapeDtypeStruct((HOP1 // IDX_W, IDX_W, POOL_T), jnp.int32),
        jax.ShapeDtypeStruct((HOP1 // IDX_W, IDX_W, POOL_T), jnp.float32),
    ),
    mesh=_MESH,
    compiler_params=pltpu.CompilerParams(use_tc_tiling_on_sc=False),
    scratch_types=[
        pltpu.VMEM((H1_CHUNKS, IDX_W), jnp.int32),
        pltpu.VMEM((H1_CHUNKS, IDX_W, POOL_T), jnp.int32),
        pltpu.VMEM((H1_CHUNKS, IDX_W, POOL_T), jnp.float32),
        pltpu.SemaphoreType.DMA,
    ],
)
def _hop1_sc(idx_hbm, neigh_hbm, alpha_hbm, neigh_out, alpha_out,
             idx_v, neigh_v, alpha_v, sem):
    wid = lax.axis_index("s") * NUM_CORES + lax.axis_index("c")
    row0 = wid * H1_CHUNKS
    pltpu.sync_copy(idx_hbm.at[pl.ds(row0, H1_CHUNKS), :], idx_v)
    copies = []
    for j in range(H1_CHUNKS):
        copies.append(
            pltpu.async_copy(neigh_hbm.at[idx_v.at[j]], neigh_v.at[j], sem))
        copies.append(
            pltpu.async_copy(alpha_hbm.at[idx_v.at[j]], alpha_v.at[j], sem))
    for cp in copies:
        cp.wait()
    pltpu.sync_copy(neigh_v, neigh_out.at[pl.ds(row0, H1_CHUNKS), :, :])
    pltpu.sync_copy(alpha_v, alpha_out.at[pl.ds(row0, H1_CHUNKS), :, :])


def kernel(node_ids, neigh_table, alpha_table):
    neigh0, alpha0 = _hop0_sc(node_ids, neigh_table, alpha_table)
    ids1 = jnp.transpose(neigh0).reshape(-1)
    neigh1, alpha1 = _hop1_sc(ids1.reshape(HOP1 // IDX_W, IDX_W),
                              neigh_table, alpha_table)
    neigh_all = jnp.concatenate([neigh1.reshape(HOP1, POOL_T), neigh0])
    alpha_all = jnp.concatenate([alpha1.reshape(HOP1, POOL_T), alpha0])
    ids = jnp.concatenate([ids1, node_ids.astype(jnp.int32)])
    stacks = jnp.concatenate([ids[:, None], neigh_all], axis=1)
    return stacks, alpha0, alpha_all
